# Initial kernel scaffold; baseline (speedup 1.0000x reference)
import jax
import jax.numpy as jnp
from jax.experimental import pallas as pl


def kernel(x, edge_index, Wsrc1, Wdst1, asrc1, adst1, b1, Wl1, bl1, Wsrc2, Wdst2, asrc2, adst2, b2, Wl2, bl2):
    raise NotImplementedError("write your pallas kernel here")


# speedup vs baseline: 13.5518x; 13.5518x over previous
import functools

import jax
import jax.numpy as jnp
from jax import lax
from jax.experimental import pallas as pl
from jax.experimental.pallas import tpu as pltpu
from jax.experimental.pallas import tpu_sc as plsc

N = 10000
D = 128
E = 320000
NEG = 0.2

NC = 2
NS = 16
NW = NC * NS
PW = 10240
EP = NW * PW
CH = 128
NCH = PW // CH
RPT = N // NS

RB = 1000
_F32 = jnp.float32

_MESH = plsc.VectorSubcoreMesh(
    core_axis_name="c", subcore_axis_name="s", num_cores=NC, num_subcores=NS)


def _worker_id():
    return lax.axis_index("s") * NC + lax.axis_index("c")


def _sc_logits_body(as_hbm, ad_hbm, src_hbm, dst_hbm, e_hbm, sall_hbm,
                    asv, adv, srcv, dstv, ev, slv):
    wid = _worker_id()
    base = wid * PW
    pltpu.sync_copy(as_hbm, asv)
    pltpu.sync_copy(ad_hbm, adv)
    pltpu.sync_copy(src_hbm.at[pl.ds(base, PW)], srcv)
    pltpu.sync_copy(dst_hbm.at[pl.ds(base, PW)], dstv)

    def zero(i, c):
        slv[pl.ds(i * 16, 16)] = jnp.zeros((16,), _F32)
        return c
    lax.fori_loop(0, N // 16, zero, 0)

    def body(i, c):
        off = i * 16
        sv = srcv[pl.ds(off, 16)]
        dv = dstv[pl.ds(off, 16)]
        lg = plsc.load_gather(asv, [sv]) + plsc.load_gather(adv, [dv])
        lg = jnp.where(lg >= 0.0, lg, lg * NEG)
        ee = jnp.exp(lg)
        gl = base + off + lax.iota(jnp.int32, 16)
        ee = jnp.where(gl < E, ee, 0.0)
        ev[pl.ds(off, 16)] = ee
        plsc.addupdate_scatter(slv, [dv], ee)
        return c
    lax.fori_loop(0, PW // 16, body, 0)

    pltpu.sync_copy(ev, e_hbm.at[pl.ds(base, PW)])
    for t in range(N // RB):
        pltpu.sync_copy(slv.at[pl.ds(t * RB, RB)], sall_hbm.at[t, wid])


_sc_logits = functools.partial(
    pl.kernel,
    out_type=(
        jax.ShapeDtypeStruct((EP,), _F32),
        jax.ShapeDtypeStruct((N // RB, NW, RB), _F32),
    ),
    mesh=_MESH,
    scratch_types=[
        pltpu.VMEM((N,), _F32),
        pltpu.VMEM((N,), _F32),
        pltpu.VMEM((PW,), jnp.int32),
        pltpu.VMEM((PW,), jnp.int32),
        pltpu.VMEM((PW,), _F32),
        pltpu.VMEM((N,), _F32),
    ],
    compiler_params=pltpu.CompilerParams(needs_layout_passes=False, use_tc_tiling_on_sc=False),
)(_sc_logits_body)


def _sc_agg_body(xs_hbm, e_hbm, src_hbm, dst_hbm, p_hbm,
                 rows, sidx, didx, ev, acc, sem):
    cid = lax.axis_index("c")
    sid = lax.axis_index("s")
    wid = sid * NC + cid
    base = wid * PW
    r0 = sid * RPT

    def zrow(i, c):
        for k in range(D // 16):
            rows[i, pl.ds(k * 16, 16)] = jnp.zeros((16,), _F32)
        return c
    lax.fori_loop(0, CH, zrow, 0)
    for off, sz in ((0, 128), (128, 128), (256, 128), (384, 128), (512, 113)):
        pltpu.sync_copy(rows.at[pl.ds(0, sz)], acc.at[pl.ds(r0 + off, sz)])
    plsc.subcore_barrier()

    def chunk(k, c):
        cb = base + k * CH
        pltpu.sync_copy(src_hbm.at[pl.ds(cb, CH)], sidx)
        pltpu.sync_copy(dst_hbm.at[pl.ds(cb, CH)], didx)
        pltpu.sync_copy(e_hbm.at[pl.ds(cb, CH)], ev)
        pltpu.async_copy(xs_hbm.at[sidx], rows, sem).wait()

        def scale(j, cc):
            a16 = ev[pl.ds(j * 16, 16)]
            for l in range(16):
                a = a16[l]
                row = j * 16 + l
                for kk in range(D // 16):
                    sl = pl.ds(kk * 16, 16)
                    rows[row, sl] = rows[row, sl] * a
            return cc
        lax.fori_loop(0, CH // 16, scale, 0)
        pltpu.sync_copy(rows, acc.at[didx], add=True)
        return c
    lax.fori_loop(0, NCH, chunk, 0)

    plsc.subcore_barrier()
    for off, sz in ((0, 128), (128, 128), (256, 128), (384, 128), (512, 113)):
        pltpu.sync_copy(acc.at[pl.ds(r0 + off, sz)],
                        p_hbm.at[pl.ds(cid * N + r0 + off, sz)])


_sc_agg = functools.partial(
    pl.kernel,
    out_type=jax.ShapeDtypeStruct((NC * N, D), _F32),
    mesh=_MESH,
    scratch_types=[
        pltpu.VMEM((CH, D), _F32),
        pltpu.VMEM((CH,), jnp.int32),
        pltpu.VMEM((CH,), jnp.int32),
        pltpu.VMEM((CH,), _F32),
        pltpu.VMEM_SHARED((N, D), _F32),
        pltpu.SemaphoreType.DMA,
    ],
    compiler_params=pltpu.CompilerParams(needs_layout_passes=False, use_tc_tiling_on_sc=False),
)(_sc_agg_body)


def _tc_head_body(x_ref, ws_ref, wd_ref, wl_ref, asr_ref, adr_ref, bl_ref,
                  xs_ref, lin_ref, as_ref, ad_ref):
    x = x_ref[...]
    xs = jnp.dot(x, ws_ref[...], preferred_element_type=_F32)
    xd = jnp.dot(x, wd_ref[...], preferred_element_type=_F32)
    xs_ref[...] = xs
    lin_ref[...] = jnp.dot(x, wl_ref[...], preferred_element_type=_F32) + bl_ref[...]
    as_ref[...] = jnp.sum(xs * asr_ref[...], axis=1)[None, None, :]
    ad_ref[...] = jnp.sum(xd * adr_ref[...], axis=1)[None, None, :]


def _tc_head(x, Wsrc, Wdst, Wl, asrc, adst, bl):
    full = pl.BlockSpec((D, D), lambda i: (0, 0))
    row = pl.BlockSpec((1, D), lambda i: (0, 0))
    blk2 = pl.BlockSpec((RB, D), lambda i: (i, 0))
    blk1 = pl.BlockSpec((1, 1, RB), lambda i: (i, 0, 0))
    return pl.pallas_call(
        _tc_head_body,
        grid=(N // RB,),
        in_specs=[blk2, full, full, full, row, row, row],
        out_specs=[blk2, blk2, blk1, blk1],
        out_shape=[
            jax.ShapeDtypeStruct((N, D), _F32),
            jax.ShapeDtypeStruct((N, D), _F32),
            jax.ShapeDtypeStruct((N // RB, 1, RB), _F32),
            jax.ShapeDtypeStruct((N // RB, 1, RB), _F32),
        ],
    )(x, Wsrc, Wdst, Wl, asrc.reshape(1, D), adst.reshape(1, D), bl.reshape(1, D))


def _tc_mid_body(p_ref, sall_ref, lin1_ref, b1_ref,
                 ws_ref, wd_ref, wl_ref, asr_ref, adr_ref, bl_ref,
                 xs_ref, lin_ref, as_ref, ad_ref):
    s = jnp.sum(sall_ref[0], axis=0)
    gat = (p_ref[0] + p_ref[1]) / (s[:, None] + 1e-16) + b1_ref[...]
    h = jax.nn.relu(gat + lin1_ref[...])
    xs = jnp.dot(h, ws_ref[...], preferred_element_type=_F32)
    xd = jnp.dot(h, wd_ref[...], preferred_element_type=_F32)
    xs_ref[...] = xs
    lin_ref[...] = jnp.dot(h, wl_ref[...], preferred_element_type=_F32) + bl_ref[...]
    as_ref[...] = jnp.sum(xs * asr_ref[...], axis=1)[None, None, :]
    ad_ref[...] = jnp.sum(xd * adr_ref[...], axis=1)[None, None, :]


def _tc_mid(p, sall, lin1, b1, Wsrc, Wdst, Wl, asrc, adst, bl):
    full = pl.BlockSpec((D, D), lambda i: (0, 0))
    row = pl.BlockSpec((1, D), lambda i: (0, 0))
    blk2 = pl.BlockSpec((RB, D), lambda i: (i, 0))
    blk1 = pl.BlockSpec((1, 1, RB), lambda i: (i, 0, 0))
    pblk = pl.BlockSpec((NC, RB, D), lambda i: (0, i, 0))
    sblk = pl.BlockSpec((1, NW, RB), lambda i: (i, 0, 0))
    return pl.pallas_call(
        _tc_mid_body,
        grid=(N // RB,),
        in_specs=[pblk, sblk, blk2, row, full, full, full, row, row, row],
        out_specs=[blk2, blk2, blk1, blk1],
        out_shape=[
            jax.ShapeDtypeStruct((N, D), _F32),
            jax.ShapeDtypeStruct((N, D), _F32),
            jax.ShapeDtypeStruct((N // RB, 1, RB), _F32),
            jax.ShapeDtypeStruct((N // RB, 1, RB), _F32),
        ],
    )(p, sall, lin1, b1.reshape(1, D),
      Wsrc, Wdst, Wl, asrc.reshape(1, D), adst.reshape(1, D), bl.reshape(1, D))


def _tc_tail_body(p_ref, sall_ref, lin2_ref, b2_ref, o_ref):
    s = jnp.sum(sall_ref[0], axis=0)
    gat = (p_ref[0] + p_ref[1]) / (s[:, None] + 1e-16) + b2_ref[...]
    o_ref[...] = jax.nn.sigmoid(gat + lin2_ref[...])


def _tc_tail(p, sall, lin2, b2):
    row = pl.BlockSpec((1, D), lambda i: (0, 0))
    blk2 = pl.BlockSpec((RB, D), lambda i: (i, 0))
    pblk = pl.BlockSpec((NC, RB, D), lambda i: (0, i, 0))
    sblk = pl.BlockSpec((1, NW, RB), lambda i: (i, 0, 0))
    return pl.pallas_call(
        _tc_tail_body,
        grid=(N // RB,),
        in_specs=[pblk, sblk, blk2, row],
        out_specs=blk2,
        out_shape=jax.ShapeDtypeStruct((N, D), _F32),
    )(p, sall, lin2, b2.reshape(1, D))


def kernel(x, edge_index, Wsrc1, Wdst1, asrc1, adst1, b1, Wl1, bl1,
           Wsrc2, Wdst2, asrc2, adst2, b2, Wl2, bl2):
    pad = EP - E
    srcp = jnp.concatenate([edge_index[0], jnp.zeros((pad,), jnp.int32)])
    dstp = jnp.concatenate([edge_index[1], jnp.zeros((pad,), jnp.int32)])

    xs1, lin1, as1, ad1 = _tc_head(x, Wsrc1, Wdst1, Wl1, asrc1, adst1, bl1)
    e1, s1 = _sc_logits(as1.reshape(N), ad1.reshape(N), srcp, dstp)
    p1 = _sc_agg(xs1, e1, srcp, dstp).reshape(NC, N, D)

    xs2, lin2, as2, ad2 = _tc_mid(p1, s1, lin1, b1,
                                  Wsrc2, Wdst2, Wl2, asrc2, adst2, bl2)
    e2, s2 = _sc_logits(as2.reshape(N), ad2.reshape(N), srcp, dstp)
    p2 = _sc_agg(xs2, e2, srcp, dstp).reshape(NC, N, D)

    return _tc_tail(p2, s2, lin2, b2)


# speedup vs baseline: 18.9117x; 1.3955x over previous
import functools

import jax
import jax.numpy as jnp
from jax import lax
from jax.experimental import pallas as pl
from jax.experimental.pallas import tpu as pltpu
from jax.experimental.pallas import tpu_sc as plsc

N = 10000
D = 128
E = 320000
NEG = 0.2

NC = 2
NS = 16
NW = NC * NS
PW = 10240
EP = NW * PW
CH = 128
NCH = PW // CH
RPT = N // NS

RB = 1000
_F32 = jnp.float32

_MESH = plsc.VectorSubcoreMesh(
    core_axis_name="c", subcore_axis_name="s", num_cores=NC, num_subcores=NS)


def _worker_id():
    return lax.axis_index("s") * NC + lax.axis_index("c")


def _sc_logits_body(as_hbm, ad_hbm, src_hbm, dst_hbm, e_hbm, sall_hbm,
                    asv, adv, srcv, dstv, ev, slv):
    wid = _worker_id()
    base = wid * PW
    pltpu.sync_copy(as_hbm, asv)
    pltpu.sync_copy(ad_hbm, adv)
    pltpu.sync_copy(src_hbm.at[pl.ds(base, PW)], srcv)
    pltpu.sync_copy(dst_hbm.at[pl.ds(base, PW)], dstv)

    def zero(i, c):
        slv[pl.ds(i * 16, 16)] = jnp.zeros((16,), _F32)
        return c
    lax.fori_loop(0, N // 16, zero, 0)

    def body(i, c):
        off = i * 16
        sv = srcv[pl.ds(off, 16)]
        dv = dstv[pl.ds(off, 16)]
        lg = plsc.load_gather(asv, [sv]) + plsc.load_gather(adv, [dv])
        lg = jnp.where(lg >= 0.0, lg, lg * NEG)
        ee = jnp.exp(lg)
        gl = base + off + lax.iota(jnp.int32, 16)
        ee = jnp.where(gl < E, ee, 0.0)
        ev[pl.ds(off, 16)] = ee
        plsc.addupdate_scatter(slv, [dv], ee)
        return c
    lax.fori_loop(0, PW // 16, body, 0)

    pltpu.sync_copy(ev, e_hbm.at[pl.ds(base, PW)])
    for t in range(N // RB):
        pltpu.sync_copy(slv.at[pl.ds(t * RB, RB)], sall_hbm.at[t, wid])


_sc_logits = functools.partial(
    pl.kernel,
    out_type=(
        jax.ShapeDtypeStruct((EP,), _F32),
        jax.ShapeDtypeStruct((N // RB, NW, RB), _F32),
    ),
    mesh=_MESH,
    scratch_types=[
        pltpu.VMEM((N,), _F32),
        pltpu.VMEM((N,), _F32),
        pltpu.VMEM((PW,), jnp.int32),
        pltpu.VMEM((PW,), jnp.int32),
        pltpu.VMEM((PW,), _F32),
        pltpu.VMEM((N,), _F32),
    ],
    compiler_params=pltpu.CompilerParams(needs_layout_passes=False, use_tc_tiling_on_sc=False),
)(_sc_logits_body)


def _sc_agg_body(xs_hbm, e2_hbm, src2_hbm, dst2_hbm, p_hbm,
                 rows0, rows1, sidx, didx, ev, acc, sem0, sem1):
    cid = lax.axis_index("c")
    sid = lax.axis_index("s")
    wid = sid * NC + cid
    r0 = sid * RPT
    crow = wid * NCH

    pltpu.sync_copy(src2_hbm.at[pl.ds(crow, NCH)], sidx)

    def zrow(i, c):
        for k in range(D // 16):
            rows0[i, pl.ds(k * 16, 16)] = jnp.zeros((16,), _F32)
        return c
    lax.fori_loop(0, CH, zrow, 0)
    for off, sz in ((0, 128), (128, 128), (256, 128), (384, 128), (512, 113)):
        pltpu.sync_copy(rows0.at[pl.ds(0, sz)], acc.at[pl.ds(r0 + off, sz)])
    plsc.subcore_barrier()

    def issue(k, b, rows, sem):
        pltpu.async_copy(dst2_hbm.at[crow + k], didx.at[b], sem)
        pltpu.async_copy(e2_hbm.at[crow + k], ev.at[b], sem)
        pltpu.async_copy(xs_hbm.at[sidx.at[k]], rows, sem)

    def wait(b, rows, sem):
        pltpu.make_async_copy(dst2_hbm.at[0], didx.at[b], sem).wait()
        pltpu.make_async_copy(e2_hbm.at[0], ev.at[b], sem).wait()
        pltpu.make_async_copy(xs_hbm.at[pl.ds(0, CH)], rows, sem).wait()

    def process(b, rows):
        def scale(j, cc):
            a16 = ev[b, pl.ds(j * 16, 16)]
            for l in range(16):
                a = a16[l]
                row = j * 16 + l
                for kk in range(D // 16):
                    sl = pl.ds(kk * 16, 16)
                    rows[row, sl] = rows[row, sl] * a
            return cc
        lax.fori_loop(0, CH // 16, scale, 0)
        pltpu.sync_copy(rows, acc.at[didx.at[b]], add=True)

    issue(0, 0, rows0, sem0)
    issue(1, 1, rows1, sem1)

    def pair(i, c):
        k = 2 * i
        wait(0, rows0, sem0)
        process(0, rows0)
        issue(k + 2, 0, rows0, sem0)
        wait(1, rows1, sem1)
        process(1, rows1)
        issue(k + 3, 1, rows1, sem1)
        return c
    lax.fori_loop(0, NCH // 2 - 1, pair, 0)

    wait(0, rows0, sem0)
    process(0, rows0)
    wait(1, rows1, sem1)
    process(1, rows1)

    plsc.subcore_barrier()
    for off, sz in ((0, 128), (128, 128), (256, 128), (384, 128), (512, 113)):
        pltpu.sync_copy(acc.at[pl.ds(r0 + off, sz)],
                        p_hbm.at[pl.ds(cid * N + r0 + off, sz)])


_sc_agg = functools.partial(
    pl.kernel,
    out_type=jax.ShapeDtypeStruct((NC * N, D), _F32),
    mesh=_MESH,
    scratch_types=[
        pltpu.VMEM((CH, D), _F32),
        pltpu.VMEM((CH, D), _F32),
        pltpu.VMEM((NCH, CH), jnp.int32),
        pltpu.VMEM((2, CH), jnp.int32),
        pltpu.VMEM((2, CH), _F32),
        pltpu.VMEM_SHARED((N, D), _F32),
        pltpu.SemaphoreType.DMA,
        pltpu.SemaphoreType.DMA,
    ],
    compiler_params=pltpu.CompilerParams(needs_layout_passes=False, use_tc_tiling_on_sc=False),
)(_sc_agg_body)


def _tc_head_body(x_ref, ws_ref, wd_ref, wl_ref, asr_ref, adr_ref, bl_ref,
                  xs_ref, lin_ref, as_ref, ad_ref):
    x = x_ref[...]
    xs = jnp.dot(x, ws_ref[...], preferred_element_type=_F32)
    xd = jnp.dot(x, wd_ref[...], preferred_element_type=_F32)
    xs_ref[...] = xs
    lin_ref[...] = jnp.dot(x, wl_ref[...], preferred_element_type=_F32) + bl_ref[...]
    as_ref[...] = jnp.sum(xs * asr_ref[...], axis=1)[None, None, :]
    ad_ref[...] = jnp.sum(xd * adr_ref[...], axis=1)[None, None, :]


def _tc_head(x, Wsrc, Wdst, Wl, asrc, adst, bl):
    full = pl.BlockSpec((D, D), lambda i: (0, 0))
    row = pl.BlockSpec((1, D), lambda i: (0, 0))
    blk2 = pl.BlockSpec((RB, D), lambda i: (i, 0))
    blk1 = pl.BlockSpec((1, 1, RB), lambda i: (i, 0, 0))
    return pl.pallas_call(
        _tc_head_body,
        grid=(N // RB,),
        in_specs=[blk2, full, full, full, row, row, row],
        out_specs=[blk2, blk2, blk1, blk1],
        out_shape=[
            jax.ShapeDtypeStruct((N, D), _F32),
            jax.ShapeDtypeStruct((N, D), _F32),
            jax.ShapeDtypeStruct((N // RB, 1, RB), _F32),
            jax.ShapeDtypeStruct((N // RB, 1, RB), _F32),
        ],
    )(x, Wsrc, Wdst, Wl, asrc.reshape(1, D), adst.reshape(1, D), bl.reshape(1, D))


def _tc_mid_body(p_ref, sall_ref, lin1_ref, b1_ref,
                 ws_ref, wd_ref, wl_ref, asr_ref, adr_ref, bl_ref,
                 xs_ref, lin_ref, as_ref, ad_ref):
    s = jnp.sum(sall_ref[0], axis=0)
    gat = (p_ref[0] + p_ref[1]) / (s[:, None] + 1e-16) + b1_ref[...]
    h = jax.nn.relu(gat + lin1_ref[...])
    xs = jnp.dot(h, ws_ref[...], preferred_element_type=_F32)
    xd = jnp.dot(h, wd_ref[...], preferred_element_type=_F32)
    xs_ref[...] = xs
    lin_ref[...] = jnp.dot(h, wl_ref[...], preferred_element_type=_F32) + bl_ref[...]
    as_ref[...] = jnp.sum(xs * asr_ref[...], axis=1)[None, None, :]
    ad_ref[...] = jnp.sum(xd * adr_ref[...], axis=1)[None, None, :]


def _tc_mid(p, sall, lin1, b1, Wsrc, Wdst, Wl, asrc, adst, bl):
    full = pl.BlockSpec((D, D), lambda i: (0, 0))
    row = pl.BlockSpec((1, D), lambda i: (0, 0))
    blk2 = pl.BlockSpec((RB, D), lambda i: (i, 0))
    blk1 = pl.BlockSpec((1, 1, RB), lambda i: (i, 0, 0))
    pblk = pl.BlockSpec((NC, RB, D), lambda i: (0, i, 0))
    sblk = pl.BlockSpec((1, NW, RB), lambda i: (i, 0, 0))
    return pl.pallas_call(
        _tc_mid_body,
        grid=(N // RB,),
        in_specs=[pblk, sblk, blk2, row, full, full, full, row, row, row],
        out_specs=[blk2, blk2, blk1, blk1],
        out_shape=[
            jax.ShapeDtypeStruct((N, D), _F32),
            jax.ShapeDtypeStruct((N, D), _F32),
            jax.ShapeDtypeStruct((N // RB, 1, RB), _F32),
            jax.ShapeDtypeStruct((N // RB, 1, RB), _F32),
        ],
    )(p, sall, lin1, b1.reshape(1, D),
      Wsrc, Wdst, Wl, asrc.reshape(1, D), adst.reshape(1, D), bl.reshape(1, D))


def _tc_tail_body(p_ref, sall_ref, lin2_ref, b2_ref, o_ref):
    s = jnp.sum(sall_ref[0], axis=0)
    gat = (p_ref[0] + p_ref[1]) / (s[:, None] + 1e-16) + b2_ref[...]
    o_ref[...] = jax.nn.sigmoid(gat + lin2_ref[...])


def _tc_tail(p, sall, lin2, b2):
    row = pl.BlockSpec((1, D), lambda i: (0, 0))
    blk2 = pl.BlockSpec((RB, D), lambda i: (i, 0))
    pblk = pl.BlockSpec((NC, RB, D), lambda i: (0, i, 0))
    sblk = pl.BlockSpec((1, NW, RB), lambda i: (i, 0, 0))
    return pl.pallas_call(
        _tc_tail_body,
        grid=(N // RB,),
        in_specs=[pblk, sblk, blk2, row],
        out_specs=blk2,
        out_shape=jax.ShapeDtypeStruct((N, D), _F32),
    )(p, sall, lin2, b2.reshape(1, D))


def kernel(x, edge_index, Wsrc1, Wdst1, asrc1, adst1, b1, Wl1, bl1,
           Wsrc2, Wdst2, asrc2, adst2, b2, Wl2, bl2):
    pad = EP - E
    srcp = jnp.concatenate([edge_index[0], jnp.zeros((pad,), jnp.int32)])
    dstp = jnp.concatenate([edge_index[1], jnp.zeros((pad,), jnp.int32)])
    src2 = srcp.reshape(EP // CH, CH)
    dst2 = dstp.reshape(EP // CH, CH)

    xs1, lin1, as1, ad1 = _tc_head(x, Wsrc1, Wdst1, Wl1, asrc1, adst1, bl1)
    e1, s1 = _sc_logits(as1.reshape(N), ad1.reshape(N), srcp, dstp)
    p1 = _sc_agg(xs1, e1.reshape(EP // CH, CH), src2, dst2).reshape(NC, N, D)

    xs2, lin2, as2, ad2 = _tc_mid(p1, s1, lin1, b1,
                                  Wsrc2, Wdst2, Wl2, asrc2, adst2, bl2)
    e2, s2 = _sc_logits(as2.reshape(N), ad2.reshape(N), srcp, dstp)
    p2 = _sc_agg(xs2, e2.reshape(EP // CH, CH), src2, dst2).reshape(NC, N, D)

    return _tc_tail(p2, s2, lin2, b2)


# speedup vs baseline: 20.4252x; 1.0800x over previous
import functools

import jax
import jax.numpy as jnp
from jax import lax
from jax.experimental import pallas as pl
from jax.experimental.pallas import tpu as pltpu
from jax.experimental.pallas import tpu_sc as plsc

N = 10000
D = 128
E = 320000
NEG = 0.2

NC = 2
NS = 16
NW = NC * NS
PW = 10240
EP = NW * PW
CH = 128
NCH = PW // CH
RPT = N // NS
D2 = D // 2
SLABS = ((0, 128), (128, 128), (256, 128), (384, 128), (512, 113))

RB = 1000
_F32 = jnp.float32

_MESH = plsc.VectorSubcoreMesh(
    core_axis_name="c", subcore_axis_name="s", num_cores=NC, num_subcores=NS)


def _worker_id():
    return lax.axis_index("s") * NC + lax.axis_index("c")


def _sc_logits_body(as_hbm, ad_hbm, src_hbm, dst_hbm, e_hbm, sall_hbm,
                    asv, adv, srcv, dstv, ev, slv):
    wid = _worker_id()
    base = wid * PW
    pltpu.sync_copy(as_hbm, asv)
    pltpu.sync_copy(ad_hbm, adv)
    pltpu.sync_copy(src_hbm.at[pl.ds(base, PW)], srcv)
    pltpu.sync_copy(dst_hbm.at[pl.ds(base, PW)], dstv)

    def zero(i, c):
        slv[pl.ds(i * 16, 16)] = jnp.zeros((16,), _F32)
        return c
    lax.fori_loop(0, N // 16, zero, 0)

    def body(i, c):
        off = i * 16
        sv = srcv[pl.ds(off, 16)]
        dv = dstv[pl.ds(off, 16)]
        lg = plsc.load_gather(asv, [sv]) + plsc.load_gather(adv, [dv])
        lg = jnp.where(lg >= 0.0, lg, lg * NEG)
        ee = jnp.exp(lg)
        gl = base + off + lax.iota(jnp.int32, 16)
        ee = jnp.where(gl < E, ee, 0.0)
        ev[pl.ds(off, 16)] = ee
        plsc.addupdate_scatter(slv, [dv], ee)
        return c
    lax.fori_loop(0, PW // 16, body, 0)

    pltpu.sync_copy(ev, e_hbm.at[pl.ds(base, PW)])
    for t in range(N // RB):
        pltpu.sync_copy(slv.at[pl.ds(t * RB, RB)], sall_hbm.at[t, wid])


_sc_logits = functools.partial(
    pl.kernel,
    out_type=(
        jax.ShapeDtypeStruct((EP,), _F32),
        jax.ShapeDtypeStruct((N // RB, NW, RB), _F32),
    ),
    mesh=_MESH,
    scratch_types=[
        pltpu.VMEM((N,), _F32),
        pltpu.VMEM((N,), _F32),
        pltpu.VMEM((PW,), jnp.int32),
        pltpu.VMEM((PW,), jnp.int32),
        pltpu.VMEM((PW,), _F32),
        pltpu.VMEM((N,), _F32),
    ],
    compiler_params=pltpu.CompilerParams(needs_layout_passes=False, use_tc_tiling_on_sc=False),
)(_sc_logits_body)


def _sc_agg_body(xs_hbm, e2_hbm, src2_hbm, dst2_hbm, p_hbm,
                 rows0, rows1, sidx, didx, ev, xsp, acc, sem0, sem1):
    cid = lax.axis_index("c")
    sid = lax.axis_index("s")
    wid = sid * NC + cid
    r0 = sid * RPT
    crow = wid * NCH

    pltpu.sync_copy(src2_hbm.at[pl.ds(crow, NCH)], sidx)
    pltpu.sync_copy(dst2_hbm.at[pl.ds(crow, NCH)], didx)
    pltpu.sync_copy(e2_hbm.at[pl.ds(crow, NCH)], ev)

    def issue(k, rows, sem):
        pltpu.async_copy(xsp.at[sidx.at[k]], rows, sem)

    def wait(rows, sem):
        pltpu.make_async_copy(xsp.at[pl.ds(0, CH)], rows, sem).wait()

    def process(b, rows):
        def scale(j, cc):
            a16 = ev[b, pl.ds(j * 16, 16)]
            for l in range(16):
                a = a16[l]
                row = j * 16 + l
                for kk in range(D2 // 16):
                    sl = pl.ds(kk * 16, 16)
                    rows[row, sl] = rows[row, sl] * a
            return cc
        lax.fori_loop(0, CH // 16, scale, 0)
        pltpu.sync_copy(rows, acc.at[didx.at[b]], add=True)

    for half in range(2):
        def zrow(i, c):
            for k in range(D2 // 16):
                rows0[i, pl.ds(k * 16, 16)] = jnp.zeros((16,), _F32)
            return c
        lax.fori_loop(0, CH, zrow, 0)
        for off, sz in SLABS:
            pltpu.sync_copy(xs_hbm.at[pl.ds(half * N + r0 + off, sz)],
                            xsp.at[pl.ds(r0 + off, sz)])
            pltpu.sync_copy(rows0.at[pl.ds(0, sz)], acc.at[pl.ds(r0 + off, sz)])
        plsc.subcore_barrier()

        issue(0, rows0, sem0)
        issue(1, rows1, sem1)

        def pair(i, c):
            k = 2 * i
            wait(rows0, sem0)

            def scale0(j, cc):
                a16 = ev[k, pl.ds(j * 16, 16)]
                for l in range(16):
                    a = a16[l]
                    row = j * 16 + l
                    for kk in range(D2 // 16):
                        sl = pl.ds(kk * 16, 16)
                        rows0[row, sl] = rows0[row, sl] * a
                return cc
            lax.fori_loop(0, CH // 16, scale0, 0)
            pltpu.sync_copy(rows0, acc.at[didx.at[k]], add=True)
            issue(k + 2, rows0, sem0)

            wait(rows1, sem1)

            def scale1(j, cc):
                a16 = ev[k + 1, pl.ds(j * 16, 16)]
                for l in range(16):
                    a = a16[l]
                    row = j * 16 + l
                    for kk in range(D2 // 16):
                        sl = pl.ds(kk * 16, 16)
                        rows1[row, sl] = rows1[row, sl] * a
                return cc
            lax.fori_loop(0, CH // 16, scale1, 0)
            pltpu.sync_copy(rows1, acc.at[didx.at[k + 1]], add=True)
            issue(k + 3, rows1, sem1)
            return c
        lax.fori_loop(0, NCH // 2 - 1, pair, 0)

        wait(rows0, sem0)
        process(NCH - 2, rows0)
        wait(rows1, sem1)
        process(NCH - 1, rows1)

        plsc.subcore_barrier()
        for off, sz in SLABS:
            pltpu.sync_copy(
                acc.at[pl.ds(r0 + off, sz)],
                p_hbm.at[pl.ds(half * (NC * N) + cid * N + r0 + off, sz)])


_sc_agg = functools.partial(
    pl.kernel,
    out_type=jax.ShapeDtypeStruct((2 * NC * N, D2), _F32),
    mesh=_MESH,
    scratch_types=[
        pltpu.VMEM((CH, D2), _F32),
        pltpu.VMEM((CH, D2), _F32),
        pltpu.VMEM((NCH, CH), jnp.int32),
        pltpu.VMEM((NCH, CH), jnp.int32),
        pltpu.VMEM((NCH, CH), _F32),
        pltpu.VMEM_SHARED((N, D2), _F32),
        pltpu.VMEM_SHARED((N, D2), _F32),
        pltpu.SemaphoreType.DMA,
        pltpu.SemaphoreType.DMA,
    ],
    compiler_params=pltpu.CompilerParams(needs_layout_passes=False, use_tc_tiling_on_sc=False),
)(_sc_agg_body)


def _tc_head_body(x_ref, ws_ref, wd_ref, wl_ref, asr_ref, adr_ref, bl_ref,
                  xs_ref, lin_ref, as_ref, ad_ref):
    x = x_ref[...]
    xs = jnp.dot(x, ws_ref[...], preferred_element_type=_F32)
    xd = jnp.dot(x, wd_ref[...], preferred_element_type=_F32)
    xs_ref[0] = xs[:, :D2]
    xs_ref[1] = xs[:, D2:]
    lin_ref[...] = jnp.dot(x, wl_ref[...], preferred_element_type=_F32) + bl_ref[...]
    as_ref[...] = jnp.sum(xs * asr_ref[...], axis=1)[None, None, :]
    ad_ref[...] = jnp.sum(xd * adr_ref[...], axis=1)[None, None, :]


def _tc_head(x, Wsrc, Wdst, Wl, asrc, adst, bl):
    full = pl.BlockSpec((D, D), lambda i: (0, 0))
    row = pl.BlockSpec((1, D), lambda i: (0, 0))
    blk2 = pl.BlockSpec((RB, D), lambda i: (i, 0))
    blk1 = pl.BlockSpec((1, 1, RB), lambda i: (i, 0, 0))
    blkh = pl.BlockSpec((2, RB, D2), lambda i: (0, i, 0))
    return pl.pallas_call(
        _tc_head_body,
        grid=(N // RB,),
        in_specs=[blk2, full, full, full, row, row, row],
        out_specs=[blkh, blk2, blk1, blk1],
        out_shape=[
            jax.ShapeDtypeStruct((2, N, D2), _F32),
            jax.ShapeDtypeStruct((N, D), _F32),
            jax.ShapeDtypeStruct((N // RB, 1, RB), _F32),
            jax.ShapeDtypeStruct((N // RB, 1, RB), _F32),
        ],
    )(x, Wsrc, Wdst, Wl, asrc.reshape(1, D), adst.reshape(1, D), bl.reshape(1, D))


def _tc_mid_body(p_ref, sall_ref, lin1_ref, b1_ref,
                 ws_ref, wd_ref, wl_ref, asr_ref, adr_ref, bl_ref,
                 xs_ref, lin_ref, as_ref, ad_ref):
    s = jnp.sum(sall_ref[0], axis=0)
    num = jnp.concatenate(
        [p_ref[0, 0] + p_ref[0, 1], p_ref[1, 0] + p_ref[1, 1]], axis=1)
    gat = num / (s[:, None] + 1e-16) + b1_ref[...]
    h = jax.nn.relu(gat + lin1_ref[...])
    xs = jnp.dot(h, ws_ref[...], preferred_element_type=_F32)
    xd = jnp.dot(h, wd_ref[...], preferred_element_type=_F32)
    xs_ref[0] = xs[:, :D2]
    xs_ref[1] = xs[:, D2:]
    lin_ref[...] = jnp.dot(h, wl_ref[...], preferred_element_type=_F32) + bl_ref[...]
    as_ref[...] = jnp.sum(xs * asr_ref[...], axis=1)[None, None, :]
    ad_ref[...] = jnp.sum(xd * adr_ref[...], axis=1)[None, None, :]


def _tc_mid(p, sall, lin1, b1, Wsrc, Wdst, Wl, asrc, adst, bl):
    full = pl.BlockSpec((D, D), lambda i: (0, 0))
    row = pl.BlockSpec((1, D), lambda i: (0, 0))
    blk2 = pl.BlockSpec((RB, D), lambda i: (i, 0))
    blk1 = pl.BlockSpec((1, 1, RB), lambda i: (i, 0, 0))
    blkh = pl.BlockSpec((2, RB, D2), lambda i: (0, i, 0))
    pblk = pl.BlockSpec((2, NC, RB, D2), lambda i: (0, 0, i, 0))
    sblk = pl.BlockSpec((1, NW, RB), lambda i: (i, 0, 0))
    return pl.pallas_call(
        _tc_mid_body,
        grid=(N // RB,),
        in_specs=[pblk, sblk, blk2, row, full, full, full, row, row, row],
        out_specs=[blkh, blk2, blk1, blk1],
        out_shape=[
            jax.ShapeDtypeStruct((2, N, D2), _F32),
            jax.ShapeDtypeStruct((N, D), _F32),
            jax.ShapeDtypeStruct((N // RB, 1, RB), _F32),
            jax.ShapeDtypeStruct((N // RB, 1, RB), _F32),
        ],
    )(p, sall, lin1, b1.reshape(1, D),
      Wsrc, Wdst, Wl, asrc.reshape(1, D), adst.reshape(1, D), bl.reshape(1, D))


def _tc_tail_body(p_ref, sall_ref, lin2_ref, b2_ref, o_ref):
    s = jnp.sum(sall_ref[0], axis=0)
    num = jnp.concatenate(
        [p_ref[0, 0] + p_ref[0, 1], p_ref[1, 0] + p_ref[1, 1]], axis=1)
    gat = num / (s[:, None] + 1e-16) + b2_ref[...]
    o_ref[...] = jax.nn.sigmoid(gat + lin2_ref[...])


def _tc_tail(p, sall, lin2, b2):
    row = pl.BlockSpec((1, D), lambda i: (0, 0))
    blk2 = pl.BlockSpec((RB, D), lambda i: (i, 0))
    pblk = pl.BlockSpec((2, NC, RB, D2), lambda i: (0, 0, i, 0))
    sblk = pl.BlockSpec((1, NW, RB), lambda i: (i, 0, 0))
    return pl.pallas_call(
        _tc_tail_body,
        grid=(N // RB,),
        in_specs=[pblk, sblk, blk2, row],
        out_specs=blk2,
        out_shape=jax.ShapeDtypeStruct((N, D), _F32),
    )(p, sall, lin2, b2.reshape(1, D))


def kernel(x, edge_index, Wsrc1, Wdst1, asrc1, adst1, b1, Wl1, bl1,
           Wsrc2, Wdst2, asrc2, adst2, b2, Wl2, bl2):
    pad = EP - E
    srcp = jnp.concatenate([edge_index[0], jnp.zeros((pad,), jnp.int32)])
    dstp = jnp.concatenate([edge_index[1], jnp.zeros((pad,), jnp.int32)])
    src2 = srcp.reshape(EP // CH, CH)
    dst2 = dstp.reshape(EP // CH, CH)

    xs1, lin1, as1, ad1 = _tc_head(x, Wsrc1, Wdst1, Wl1, asrc1, adst1, bl1)
    e1, s1 = _sc_logits(as1.reshape(N), ad1.reshape(N), srcp, dstp)
    p1 = _sc_agg(xs1.reshape(2 * N, D2), e1.reshape(EP // CH, CH),
                 src2, dst2).reshape(2, NC, N, D2)

    xs2, lin2, as2, ad2 = _tc_mid(p1, s1, lin1, b1,
                                  Wsrc2, Wdst2, Wl2, asrc2, adst2, bl2)
    e2, s2 = _sc_logits(as2.reshape(N), ad2.reshape(N), srcp, dstp)
    p2 = _sc_agg(xs2.reshape(2 * N, D2), e2.reshape(EP // CH, CH),
                 src2, dst2).reshape(2, NC, N, D2)

    return _tc_tail(p2, s2, lin2, b2)
